# baseline (speedup 1.0000x reference)
import functools

import jax
import jax.numpy as jnp
from jax import lax
from jax.experimental import pallas as pl
from jax.experimental.pallas import tpu as pltpu
from jax.experimental.pallas import tpu_sc as plsc

_MAX_NORM = 1.0 - 1e-5

_N = 1000000
_D = 32
_BK = 8192
_NBLK = 1007616 // _BK
_NPAD = _NBLK * _BK


def _project_pack_block(w_ref, o_ref):
    x = w_ref[...]
    norm = jnp.sqrt(jnp.sum(x * x, axis=0, keepdims=True))
    safe = jnp.maximum(norm, 1e-15)
    proj = x / safe * _MAX_NORM
    y = jnp.where(norm > _MAX_NORM, proj, x)
    for c in range(4):
        z = y[:, 2048 * c:2048 * (c + 1)].T
        o_ref[:, :, 32 * c:32 * (c + 1)] = z.reshape(256, 8, 32)


def _project_table_packed(weight_t):
    return pl.pallas_call(
        _project_pack_block,
        grid=(_NBLK,),
        in_specs=[pl.BlockSpec((_D, _BK), lambda i: (0, i))],
        out_specs=pl.BlockSpec((256, 8, 128), lambda i: (i, 0, 0)),
        out_shape=jax.ShapeDtypeStruct((_NPAD // 32, 8, 128), jnp.float32),
    )(weight_t)


_NC, _NS = 2, 16
_NW = _NC * _NS
_B = 16384 * 200
_SUB = 128
_NITEM = _B // _SUB
_IPW = _NITEM // _NW
_IPC = 8
_NCH = _IPW // _IPC


def _gather_kernel(table, idx2d, out_flat, idx_v, idxq_v, rows_v, stage_v,
                   idx_sem, g_sem, s_sem):
    wid = lax.axis_index("s") * _NC + lax.axis_index("c")
    item0 = wid * _IPW

    def idx_copy(g, b):
        off = pl.multiple_of(item0 + g * _IPC, _IPC)
        return pltpu.make_async_copy(
            idx2d.at[pl.ds(off, _IPC)], idx_v.at[b], idx_sem.at[b])

    def gather_copy(b, it):
        return pltpu.make_async_copy(
            table.at[idxq_v.at[b, it]], rows_v.at[b, it], g_sem.at[b, it])

    def scatter_copies(g):
        titem = item0 + g * _IPC
        j = jnp.right_shift(titem, 7)
        i1280 = jnp.bitwise_and(titem, 127)
        base = j * 524288 + i1280 * 1024
        seg = _IPC * 1024
        return [
            pltpu.make_async_copy(
                stage_v.at[pl.ds(t * seg, seg)],
                out_flat.at[pl.ds(pl.multiple_of(base + t * 131072, 1024),
                                  seg)],
                s_sem)
            for t in range(4)
        ]

    def transform(b):
        for it in range(_IPC):
            for k in range(8):
                v = idx_v[b, it, pl.ds(16 * k, 16)]
                m = jnp.bitwise_and(v, 2047)
                c = jnp.bitwise_and(jnp.right_shift(v, 11), 3)
                q = jnp.bitwise_or(
                    jnp.bitwise_or(jnp.bitwise_and(v, ~8191),
                                   jnp.left_shift(jnp.right_shift(m, 3), 5)),
                    jnp.bitwise_or(
                        jnp.left_shift(jnp.bitwise_and(m, 7), 2), c))
                idxq_v[b, it, pl.ds(16 * k, 16)] = q

    def transpose_item(b, it):
        iota16 = jax.lax.iota(jnp.int32, 16)
        lg = [iota16 + 16 * g for g in range(8)]
        ref = rows_v.at[b, it]
        sref = stage_v
        it_off = it * 1024

        def f(d):
            return jnp.bitwise_or(
                jnp.left_shift(jnp.right_shift(d, 3), 13),
                jnp.left_shift(jnp.bitwise_and(d, 7), 7))

        def o_body(o, carry):
            d_e = jnp.bitwise_and(iota16 + o, 31)
            d_o = jnp.bitwise_and(d_e + 16, 31)
            f_e = jnp.bitwise_or(f(d_e), it_off)
            f_o = jnp.bitwise_or(f(d_o), it_off)
            for g in range(8):
                d = d_e if g % 2 == 0 else d_o
                fd = f_e if g % 2 == 0 else f_o
                v = plsc.load_gather(ref, [lg[g], d])
                plsc.store_scatter(sref, [jnp.bitwise_or(fd, lg[g])], v)
            return carry

        lax.fori_loop(0, _D, o_body, 0)

    for b in range(2):
        idx_copy(b, b).start()
    idx_copy(0, 0).wait()
    transform(0)
    idx_copy(2, 0).start()
    for it in range(_IPC):
        gather_copy(0, it).start()

    def body(i, carry):
        gg = i * 2
        for b in range(2):
            g = gg + b
            bn = 1 - b

            @pl.when(g + 1 < _NCH)
            def _launch_next():
                idx_copy(g + 1, bn).wait()
                transform(bn)

                @pl.when(g + 3 < _NCH)
                def _prefetch_ids():
                    idx_copy(g + 3, bn).start()

                for it in range(_IPC):
                    gather_copy(bn, it).start()

            for it in range(_IPC):
                gather_copy(b, it).wait()

            @pl.when(g >= 1)
            def _wait_prev_scatter():
                for cp in scatter_copies(g - 1):
                    cp.wait()

            for it in range(_IPC):
                transpose_item(b, it)

            for cp in scatter_copies(g):
                cp.start()
        return carry

    lax.fori_loop(0, _NCH // 2, body, 0)

    for cp in scatter_copies(_NCH - 1):
        cp.wait()


def _sc_gather(ptable_rows, ids_t2d):
    mesh = plsc.VectorSubcoreMesh(core_axis_name="c", subcore_axis_name="s")
    f = pl.kernel(
        _gather_kernel,
        mesh=mesh,
        out_type=jax.ShapeDtypeStruct((_B * _D,), jnp.float32),
        scratch_types=[
            pltpu.VMEM((2, _IPC, _SUB), jnp.int32),
            pltpu.VMEM((2, _IPC, _SUB), jnp.int32),
            pltpu.VMEM((2, _IPC, _SUB, _D), jnp.float32),
            pltpu.VMEM((4 * _IPC * 1024,), jnp.float32),
            pltpu.SemaphoreType.DMA((2,)),
            pltpu.SemaphoreType.DMA((2, _IPC)),
            pltpu.SemaphoreType.DMA,
        ],
        compiler_params=pltpu.CompilerParams(use_tc_tiling_on_sc=False,
                                             needs_layout_passes=False),
    )
    return f(ptable_rows, ids_t2d)


def kernel(ids, weight):
    n, d = weight.shape
    packed = _project_table_packed(weight.T)
    ptable_rows = packed.reshape(_NPAD, _D)
    ids_t2d = ids.T.reshape(_NITEM, _SUB)
    out_flat = _sc_gather(ptable_rows, ids_t2d)
    out5 = out_flat.reshape(200, 4, 128, 8, 128)
    return out5.transpose(2, 4, 0, 1, 3).reshape(ids.shape + (d,))


# speedup vs baseline: 1.1015x; 1.1015x over previous
import functools

import jax
import jax.numpy as jnp
from jax import lax
from jax.experimental import pallas as pl
from jax.experimental.pallas import tpu as pltpu
from jax.experimental.pallas import tpu_sc as plsc

_MAX_NORM = 1.0 - 1e-5

_N = 1000000
_D = 32
_BK = 8192
_NBLK = 1007616 // _BK
_NPAD = _NBLK * _BK


def _project_pack_block(w_ref, o_ref):
    x = w_ref[...]
    norm = jnp.sqrt(jnp.sum(x * x, axis=0, keepdims=True))
    safe = jnp.maximum(norm, 1e-15)
    proj = x / safe * _MAX_NORM
    y = jnp.where(norm > _MAX_NORM, proj, x)
    for c in range(4):
        z = y[:, 2048 * c:2048 * (c + 1)].T
        o_ref[:, :, 32 * c:32 * (c + 1)] = z.reshape(256, 8, 32)


def _project_table_packed(weight_t):
    return pl.pallas_call(
        _project_pack_block,
        grid=(_NBLK,),
        in_specs=[pl.BlockSpec((_D, _BK), lambda i: (0, i))],
        out_specs=pl.BlockSpec((256, 8, 128), lambda i: (i, 0, 0)),
        out_shape=jax.ShapeDtypeStruct((_NPAD // 32, 8, 128), jnp.float32),
    )(weight_t)


_NC, _NS = 2, 16
_NW = _NC * _NS
_B = 16384 * 200
_SUB = 128
_NITEM = _B // _SUB
_IPW = _NITEM // _NW
_IPC = 4
_NCH = _IPW // _IPC


def _gather_kernel(table, idx2d, out_flat, idx_v, idxq_v, rows_v, stage_v,
                   idx_sem, g_sem, s_sem):
    wid = lax.axis_index("s") * _NC + lax.axis_index("c")
    item0 = wid * _IPW

    def idx_copy(g, b):
        off = pl.multiple_of(item0 + g * _IPC, _IPC)
        return pltpu.make_async_copy(
            idx2d.at[pl.ds(off, _IPC)], idx_v.at[b], idx_sem.at[b])

    def gather_copy(b, it):
        return pltpu.make_async_copy(
            table.at[idxq_v.at[b, it]], rows_v.at[b, it], g_sem.at[b, it])

    def scatter_copies(g, b):
        titem = item0 + g * _IPC
        j = jnp.right_shift(titem, 7)
        i1280 = jnp.bitwise_and(titem, 127)
        base = j * 524288 + i1280 * 1024
        return [
            pltpu.make_async_copy(
                stage_v.at[b, pl.ds(t * 4096, 4096)],
                out_flat.at[pl.ds(pl.multiple_of(base + t * 131072, 1024),
                                  4096)],
                s_sem.at[b])
            for t in range(4)
        ]

    def transform(b):
        for it in range(_IPC):
            for k in range(8):
                v = idx_v[b, it, pl.ds(16 * k, 16)]
                m = jnp.bitwise_and(v, 2047)
                c = jnp.bitwise_and(jnp.right_shift(v, 11), 3)
                q = jnp.bitwise_or(
                    jnp.bitwise_or(jnp.bitwise_and(v, ~8191),
                                   jnp.left_shift(jnp.right_shift(m, 3), 5)),
                    jnp.bitwise_or(
                        jnp.left_shift(jnp.bitwise_and(m, 7), 2), c))
                idxq_v[b, it, pl.ds(16 * k, 16)] = q

    def transpose_item(b, it):
        iota16 = jax.lax.iota(jnp.int32, 16)
        lg = [iota16 + 16 * g for g in range(8)]
        ref = rows_v.at[b, it]
        sref = stage_v.at[b]
        it_off = it * 1024

        def f(d):
            return jnp.bitwise_or(
                jnp.left_shift(jnp.right_shift(d, 3), 12),
                jnp.left_shift(jnp.bitwise_and(d, 7), 7))

        def o_body(o, carry):
            d_e = jnp.bitwise_and(iota16 + o, 31)
            d_o = jnp.bitwise_and(d_e + 16, 31)
            f_e = jnp.bitwise_or(f(d_e), it_off)
            f_o = jnp.bitwise_or(f(d_o), it_off)
            for g in range(8):
                d = d_e if g % 2 == 0 else d_o
                fd = f_e if g % 2 == 0 else f_o
                v = plsc.load_gather(ref, [lg[g], d])
                plsc.store_scatter(sref, [jnp.bitwise_or(fd, lg[g])], v)
            return carry

        lax.fori_loop(0, _D, o_body, 0)

    for b in range(2):
        idx_copy(b, b).start()
    idx_copy(0, 0).wait()
    transform(0)
    idx_copy(2, 0).start()
    for it in range(_IPC):
        gather_copy(0, it).start()

    def body(i, carry):
        gg = i * 2
        for b in range(2):
            g = gg + b
            bn = 1 - b

            @pl.when(g + 1 < _NCH)
            def _launch_next():
                idx_copy(g + 1, bn).wait()
                transform(bn)

                @pl.when(g + 3 < _NCH)
                def _prefetch_ids():
                    idx_copy(g + 3, bn).start()

                for it in range(_IPC):
                    gather_copy(bn, it).start()

            for it in range(_IPC):
                gather_copy(b, it).wait()

            @pl.when(g >= 2)
            def _wait_prev_scatter():
                for cp in scatter_copies(g - 2, b):
                    cp.wait()

            for it in range(_IPC):
                transpose_item(b, it)

            for cp in scatter_copies(g, b):
                cp.start()
        return carry

    lax.fori_loop(0, _NCH // 2, body, 0)

    for b in range(2):
        for cp in scatter_copies(_NCH - 2 + b, b):
            cp.wait()


def _sc_gather(ptable_rows, ids_t2d):
    mesh = plsc.VectorSubcoreMesh(core_axis_name="c", subcore_axis_name="s")
    f = pl.kernel(
        _gather_kernel,
        mesh=mesh,
        out_type=jax.ShapeDtypeStruct((_B * _D,), jnp.float32),
        scratch_types=[
            pltpu.VMEM((2, _IPC, _SUB), jnp.int32),
            pltpu.VMEM((2, _IPC, _SUB), jnp.int32),
            pltpu.VMEM((2, _IPC, _SUB, _D), jnp.float32),
            pltpu.VMEM((2, 4 * _IPC * 8 * 128), jnp.float32),
            pltpu.SemaphoreType.DMA((2,)),
            pltpu.SemaphoreType.DMA((2, _IPC)),
            pltpu.SemaphoreType.DMA((2,)),
        ],
        compiler_params=pltpu.CompilerParams(use_tc_tiling_on_sc=False,
                                             needs_layout_passes=False),
    )
    return f(ptable_rows, ids_t2d)


def kernel(ids, weight):
    n, d = weight.shape
    packed = _project_table_packed(weight.T)
    ptable_rows = packed.reshape(_NPAD, _D)
    ids_t2d = ids.T.reshape(_NITEM, _SUB)
    out_flat = _sc_gather(ptable_rows, ids_t2d)
    out5 = out_flat.reshape(200, 4, 128, 8, 128)
    return out5.transpose(2, 4, 0, 1, 3).reshape(ids.shape + (d,))
